# Initial kernel scaffold; baseline (speedup 1.0000x reference)
import functools

import jax
import jax.numpy as jnp
from jax import lax
from jax.experimental import pallas as pl
from jax.experimental.pallas import tpu as pltpu
from jax.experimental.pallas import tpu_sc as plsc

N_ROWS = 320000
D = 128
N_SEG = 10000
NC = 2
NS = 16
NW = NC * NS
CHUNK = 256
N_CHUNKS = N_ROWS // CHUNK
CHUNKS_PER_W = -(-N_CHUNKS // NW)
SEG_PER_TILE = N_SEG // NS


def _sc_partial_sums(data, ids2d):
    mesh = plsc.VectorSubcoreMesh(
        core_axis_name="c", subcore_axis_name="s", num_cores=NC, num_subcores=NS
    )

    @functools.partial(
        pl.kernel,
        out_type=jax.ShapeDtypeStruct((NC, N_SEG, D), jnp.float32),
        mesh=mesh,
        scratch_types=[
            pltpu.VMEM((CHUNK, D), jnp.float32),
            pltpu.VMEM((CHUNK // D, D), jnp.int32),
            pltpu.VMEM_SHARED((N_SEG, D), jnp.float32),
        ],
    )
    def seg_sum_kernel(data_hbm, ids_hbm, part_hbm, db, ib, acc):
        c = lax.axis_index("c")
        s = lax.axis_index("s")
        w = c * NS + s

        zeros16 = jnp.zeros((16,), jnp.float32)

        def zrow(r, carry):
            def zlane(l, cc):
                db[r, pl.ds(l * 16, 16)] = zeros16
                return cc
            return lax.fori_loop(0, D // 16, zlane, carry)

        lax.fori_loop(0, CHUNK, zrow, 0)
        base = s * SEG_PER_TILE
        pltpu.sync_copy(db.at[pl.ds(0, 256)], acc.at[pl.ds(base, 256)])
        pltpu.sync_copy(db.at[pl.ds(0, 256)], acc.at[pl.ds(base + 256, 256)])
        pltpu.sync_copy(db.at[pl.ds(0, 113)], acc.at[pl.ds(base + 512, 113)])
        plsc.subcore_barrier()

        n_mine = jnp.minimum(CHUNKS_PER_W, N_CHUNKS - w * CHUNKS_PER_W)

        def body(j, carry):
            g = w * CHUNKS_PER_W + j
            pltpu.sync_copy(data_hbm.at[pl.ds(g * CHUNK, CHUNK)], db)
            pltpu.sync_copy(ids_hbm.at[pl.ds(g * (CHUNK // D), CHUNK // D)], ib)
            pltpu.sync_copy(db.at[pl.ds(0, D)], acc.at[ib.at[0]], add=True)
            pltpu.sync_copy(db.at[pl.ds(D, D)], acc.at[ib.at[1]], add=True)
            return carry

        lax.fori_loop(0, n_mine, body, 0)
        plsc.subcore_barrier()

        pltpu.sync_copy(acc.at[pl.ds(base, 256)], part_hbm.at[c, pl.ds(base, 256)])
        pltpu.sync_copy(
            acc.at[pl.ds(base + 256, 256)], part_hbm.at[c, pl.ds(base + 256, 256)]
        )
        pltpu.sync_copy(
            acc.at[pl.ds(base + 512, 113)], part_hbm.at[c, pl.ds(base + 512, 113)]
        )

    return seg_sum_kernel(data, ids2d)


def _combine_body(p_ref, o_ref):
    o_ref[...] = p_ref[0] + p_ref[1]


def _tc_combine(partials):
    blk = 1000
    return pl.pallas_call(
        _combine_body,
        out_shape=jax.ShapeDtypeStruct((N_SEG, D), jnp.float32),
        grid=(N_SEG // blk,),
        in_specs=[pl.BlockSpec((NC, blk, D), lambda i: (0, i, 0))],
        out_specs=pl.BlockSpec((blk, D), lambda i: (i, 0)),
    )(partials)


@jax.jit
def kernel(data, segment_ids):
    ids2d = segment_ids.reshape(N_ROWS // D, D)
    partials = _sc_partial_sums(data, ids2d)
    return _tc_combine(partials)


# speedup vs baseline: 3.9228x; 3.9228x over previous
import functools

import jax
import jax.numpy as jnp
from jax import lax
from jax.experimental import pallas as pl
from jax.experimental.pallas import tpu as pltpu
from jax.experimental.pallas import tpu_sc as plsc

N_ROWS = 320000
D = 128
N_SEG = 10000
NC = 2
NS = 16
NW = NC * NS
BLOCK = 1024
PIECE = 256
N_BLOCKS = N_ROWS // BLOCK
TAIL_ROWS = N_ROWS - N_BLOCKS * BLOCK
BLOCKS_PER_W = -(-N_BLOCKS // NW)
SEG_PER_TILE = 624
SEG_TAIL = N_SEG - SEG_PER_TILE * NS


def _sc_partial_sums(data, ids2d):
    mesh = plsc.VectorSubcoreMesh(
        core_axis_name="c", subcore_axis_name="s", num_cores=NC, num_subcores=NS
    )

    @functools.partial(
        pl.kernel,
        out_type=jax.ShapeDtypeStruct((NC, N_SEG, D), jnp.float32),
        mesh=mesh,
        scratch_types=[
            pltpu.VMEM((PIECE, D), jnp.float32),
            pltpu.VMEM((BLOCK // D, D), jnp.int32),
            pltpu.VMEM_SHARED((N_SEG, D), jnp.float32),
        ],
    )
    def seg_sum_kernel(data_hbm, ids_hbm, part_hbm, db, ib, acc):
        c = lax.axis_index("c")
        s = lax.axis_index("s")
        w = c * NS + s

        zeros16 = jnp.zeros((16,), jnp.float32)

        def zrow(r, carry):
            def zlane(l, cc):
                db[r, pl.ds(l * 16, 16)] = zeros16
                return cc
            return lax.fori_loop(0, D // 16, zlane, carry)

        lax.fori_loop(0, PIECE, zrow, 0)
        base = s * SEG_PER_TILE
        pltpu.sync_copy(db.at[pl.ds(0, 256)], acc.at[pl.ds(base, 256)])
        pltpu.sync_copy(db.at[pl.ds(0, 256)], acc.at[pl.ds(base + 256, 256)])
        pltpu.sync_copy(db.at[pl.ds(0, 112)], acc.at[pl.ds(base + 512, 112)])

        @pl.when(s == NS - 1)
        def _():
            pltpu.sync_copy(
                db.at[pl.ds(0, SEG_TAIL)],
                acc.at[pl.ds(NS * SEG_PER_TILE, SEG_TAIL)],
            )

        plsc.subcore_barrier()

        def do_piece(row0, idx_row):
            pltpu.sync_copy(data_hbm.at[pl.ds(row0, PIECE)], db)
            pltpu.sync_copy(db.at[pl.ds(0, D)], acc.at[ib.at[idx_row]], add=True)
            pltpu.sync_copy(
                db.at[pl.ds(D, D)], acc.at[ib.at[idx_row + 1]], add=True
            )

        n_mine = jnp.clip(N_BLOCKS - w * BLOCKS_PER_W, 0, BLOCKS_PER_W)

        def body(j, carry):
            q = w * BLOCKS_PER_W + j
            pltpu.sync_copy(ids_hbm.at[pl.ds(q * 8, 8)], ib)
            for k in range(BLOCK // PIECE):
                do_piece(q * BLOCK + k * PIECE, 2 * k)
            return carry

        lax.fori_loop(0, n_mine, body, 0)

        @pl.when(w == NW - 1)
        def _():
            pltpu.sync_copy(
                ids_hbm.at[pl.ds(N_BLOCKS * 8, TAIL_ROWS // D)],
                ib.at[pl.ds(0, TAIL_ROWS // D)],
            )
            for k in range(TAIL_ROWS // PIECE):
                do_piece(N_BLOCKS * BLOCK + k * PIECE, 2 * k)

        plsc.subcore_barrier()

        pltpu.sync_copy(
            acc.at[pl.ds(base, SEG_PER_TILE)],
            part_hbm.at[c, pl.ds(base, SEG_PER_TILE)],
        )

        @pl.when(s == NS - 1)
        def _():
            pltpu.sync_copy(
                acc.at[pl.ds(NS * SEG_PER_TILE, SEG_TAIL)],
                part_hbm.at[c, pl.ds(NS * SEG_PER_TILE, SEG_TAIL)],
            )

    return seg_sum_kernel(data, ids2d)


def _combine_body(p_ref, o_ref):
    o_ref[...] = p_ref[0] + p_ref[1]


def _tc_combine(partials):
    blk = 1000
    return pl.pallas_call(
        _combine_body,
        out_shape=jax.ShapeDtypeStruct((N_SEG, D), jnp.float32),
        grid=(N_SEG // blk,),
        in_specs=[pl.BlockSpec((NC, blk, D), lambda i: (0, i, 0))],
        out_specs=pl.BlockSpec((blk, D), lambda i: (i, 0)),
    )(partials)


@jax.jit
def kernel(data, segment_ids):
    ids2d = segment_ids.reshape(N_ROWS // D, D)
    partials = _sc_partial_sums(data, ids2d)
    return _tc_combine(partials)


# speedup vs baseline: 5.3021x; 1.3516x over previous
import functools

import jax
import jax.numpy as jnp
from jax import lax
from jax.experimental import pallas as pl
from jax.experimental.pallas import tpu as pltpu
from jax.experimental.pallas import tpu_sc as plsc

N_ROWS = 320000
D = 128
N_SEG = 10000
NC = 2
NS = 16
NW = NC * NS
PIECE = 128
N_PIECES = N_ROWS // PIECE
PIECES_PER_W = 80
IDS_PAD_ROWS = PIECES_PER_W * NW
NBUF = 2
SEG_PER_TILE = 624
SEG_TAIL = N_SEG - SEG_PER_TILE * NS


def _sc_partial_sums(data, ids2d):
    mesh = plsc.VectorSubcoreMesh(
        core_axis_name="c", subcore_axis_name="s", num_cores=NC, num_subcores=NS
    )

    @functools.partial(
        pl.kernel,
        out_type=jax.ShapeDtypeStruct((NC, N_SEG, D), jnp.float32),
        mesh=mesh,
        scratch_types=[
            pltpu.VMEM((NBUF, PIECE, D), jnp.float32),
            pltpu.VMEM((PIECES_PER_W, D), jnp.int32),
            pltpu.VMEM_SHARED((N_SEG, D), jnp.float32),
        ]
        + [pltpu.SemaphoreType.DMA] * (2 * NBUF),
    )
    def seg_sum_kernel(data_hbm, ids_hbm, part_hbm, db, ib, acc, *sems):
        sem_g = sems[:NBUF]
        sem_s = sems[NBUF:]
        c = lax.axis_index("c")
        s = lax.axis_index("s")
        w = c * NS + s

        zeros16 = jnp.zeros((16,), jnp.float32)

        def zrow(r, carry):
            def zlane(l, cc):
                db[0, r, pl.ds(l * 16, 16)] = zeros16
                return cc
            return lax.fori_loop(0, D // 16, zlane, carry)

        lax.fori_loop(0, PIECE, zrow, 0)
        base = s * SEG_PER_TILE
        for off in range(0, 512, PIECE):
            pltpu.sync_copy(db.at[0], acc.at[pl.ds(base + off, PIECE)])
        pltpu.sync_copy(db.at[0, pl.ds(0, 112)], acc.at[pl.ds(base + 512, 112)])

        @pl.when(s == NS - 1)
        def _():
            pltpu.sync_copy(
                db.at[0, pl.ds(0, SEG_TAIL)],
                acc.at[pl.ds(NS * SEG_PER_TILE, SEG_TAIL)],
            )

        plsc.subcore_barrier()

        pltpu.sync_copy(ids_hbm.at[pl.ds(w * PIECES_PER_W, PIECES_PER_W)], ib)

        n_p = jnp.clip(N_PIECES - w * PIECES_PER_W, 0, PIECES_PER_W)
        row0 = w * PIECES_PER_W * PIECE

        def start_gather(p, b):
            pltpu.async_copy(
                data_hbm.at[pl.ds(row0 + p * PIECE, PIECE)], db.at[b], sem_g[b]
            )

        def wait_gather(b):
            pltpu.make_async_copy(
                data_hbm.at[pl.ds(row0, PIECE)], db.at[b], sem_g[b]
            ).wait()

        def start_scatter(p, b):
            pltpu.async_copy(db.at[b], acc.at[ib.at[p]], sem_s[b], add=True)

        def wait_scatter(b):
            pltpu.make_async_copy(db.at[b], acc.at[ib.at[0]], sem_s[b]).wait()

        start_gather(0, 0)

        def pair(p2, carry):
            for k in range(NBUF):
                p = p2 * NBUF + k
                b2 = (k + 1) % NBUF

                @pl.when(p + 1 < n_p)
                def _(p=p, k=k, b2=b2):
                    @pl.when(p >= 1)
                    def _():
                        wait_scatter(b2)

                    start_gather(p + 1, b2)

                wait_gather(k)
                start_scatter(p, k)
            return carry

        lax.fori_loop(0, n_p // NBUF, pair, 0)
        for b in range(NBUF):
            wait_scatter(b)
        plsc.subcore_barrier()

        pltpu.sync_copy(
            acc.at[pl.ds(base, SEG_PER_TILE)],
            part_hbm.at[c, pl.ds(base, SEG_PER_TILE)],
        )

        @pl.when(s == NS - 1)
        def _():
            pltpu.sync_copy(
                acc.at[pl.ds(NS * SEG_PER_TILE, SEG_TAIL)],
                part_hbm.at[c, pl.ds(NS * SEG_PER_TILE, SEG_TAIL)],
            )

    return seg_sum_kernel(data, ids2d)


def _combine_body(p_ref, o_ref):
    o_ref[...] = p_ref[0] + p_ref[1]


def _tc_combine(partials):
    blk = 1000
    return pl.pallas_call(
        _combine_body,
        out_shape=jax.ShapeDtypeStruct((N_SEG, D), jnp.float32),
        grid=(N_SEG // blk,),
        in_specs=[pl.BlockSpec((NC, blk, D), lambda i: (0, i, 0))],
        out_specs=pl.BlockSpec((blk, D), lambda i: (i, 0)),
    )(partials)


@jax.jit
def kernel(data, segment_ids):
    ids_pad = jnp.pad(segment_ids, (0, IDS_PAD_ROWS * D - N_ROWS))
    ids2d = ids_pad.reshape(IDS_PAD_ROWS, D)
    partials = _sc_partial_sums(data, ids2d)
    return _tc_combine(partials)


# speedup vs baseline: 5.7183x; 1.0785x over previous
import functools

import jax
import jax.numpy as jnp
from jax import lax
from jax.experimental import pallas as pl
from jax.experimental.pallas import tpu as pltpu
from jax.experimental.pallas import tpu_sc as plsc

N_ROWS = 320000
D = 128
N_SEG = 10000
NC = 2
NS = 16
PIECE = 128
N_PIECES = N_ROWS // PIECE
NBUF = 2
SPLIT = N_SEG // 2
BS_ITERS = 12
ZROWS = 312
ZTAIL = N_SEG // 2 - ZROWS * NS


def _sc_segment_sum(data, ids):
    mesh = plsc.VectorSubcoreMesh(
        core_axis_name="c", subcore_axis_name="s", num_cores=NC, num_subcores=NS
    )

    @functools.partial(
        pl.kernel,
        out_type=jax.ShapeDtypeStruct((N_SEG, D), jnp.float32),
        mesh=mesh,
        scratch_types=[
            pltpu.VMEM((NBUF, PIECE, D), jnp.float32),
            pltpu.VMEM((NBUF, PIECE), jnp.int32),
            pltpu.VMEM((16,), jnp.int32),
            pltpu.VMEM_SHARED((N_SEG, D), jnp.float32),
        ]
        + [pltpu.SemaphoreType.DMA] * (3 * NBUF),
    )
    def seg_sum_kernel(data_hbm, ids_hbm, out_hbm, db, ib, sb, acc, *sems):
        sem_g = sems[:NBUF]
        sem_i = sems[NBUF : 2 * NBUF]
        sem_s = sems[2 * NBUF :]
        c = lax.axis_index("c")
        s = lax.axis_index("s")

        def probe(q):
            pltpu.sync_copy(
                ids_hbm.at[pl.ds(pl.multiple_of(q * PIECE, PIECE), 16)], sb
            )
            v = sb[...]
            return v[0] >= SPLIT

        def bs_step(_, lohi):
            lo, hi = lohi
            mid = (lo + hi) // 2
            pred = probe(mid)
            return jnp.where(pred, lo, mid), jnp.where(pred, mid, hi)

        pred0 = probe(0)
        _, qb = lax.fori_loop(0, BS_ITERS, bs_step, (0, N_PIECES))
        qb = jnp.where(pred0, 0, qb)
        sp = jnp.maximum(qb - 1, 0)

        zeros16 = jnp.zeros((16,), jnp.float32)

        def zrow(r, carry):
            def zlane(l, cc):
                db[0, r, pl.ds(l * 16, 16)] = zeros16
                return cc
            return lax.fori_loop(0, D // 16, zlane, carry)

        lax.fori_loop(0, PIECE, zrow, 0)
        zbase = c * SPLIT + s * ZROWS
        for off in range(0, ZROWS - PIECE, PIECE):
            pltpu.sync_copy(db.at[0], acc.at[pl.ds(zbase + off, PIECE)])
        pltpu.sync_copy(
            db.at[0, pl.ds(0, ZROWS - 256)], acc.at[pl.ds(zbase + 256, ZROWS - 256)]
        )

        @pl.when(s == NS - 1)
        def _():
            pltpu.sync_copy(
                db.at[0, pl.ds(0, ZTAIL)],
                acc.at[pl.ds(c * SPLIT + NS * ZROWS, ZTAIL)],
            )

        plsc.subcore_barrier()

        p_base = jnp.where(c == 0, 0, sp)
        n_sc = jnp.where(c == 0, sp + 1, N_PIECES - sp)
        n_j = (n_sc - s + NS - 1) // NS

        def row0_of(j):
            p = p_base + s + NS * j
            return pl.multiple_of(p * PIECE, PIECE)

        def start_gather(j, b):
            r0 = row0_of(j)
            pltpu.async_copy(data_hbm.at[pl.ds(r0, PIECE)], db.at[b], sem_g[b])
            pltpu.async_copy(ids_hbm.at[pl.ds(r0, PIECE)], ib.at[b], sem_i[b])

        def wait_gather(b):
            pltpu.make_async_copy(
                data_hbm.at[pl.ds(0, PIECE)], db.at[b], sem_g[b]
            ).wait()
            pltpu.make_async_copy(
                ids_hbm.at[pl.ds(0, PIECE)], ib.at[b], sem_i[b]
            ).wait()

        def start_scatter(b):
            pltpu.async_copy(db.at[b], acc.at[ib.at[b]], sem_s[b], add=True)

        def wait_scatter(b):
            pltpu.make_async_copy(db.at[b], acc.at[ib.at[0]], sem_s[b]).wait()

        @pl.when(n_j >= 1)
        def _():
            start_gather(0, 0)

        def pair(jj, carry):
            for k in range(NBUF):
                j = jj * NBUF + k
                b2 = (k + 1) % NBUF

                @pl.when(j < n_j)
                def _(j=j, k=k, b2=b2):
                    @pl.when(j + 1 < n_j)
                    def _():
                        @pl.when(j >= 1)
                        def _():
                            wait_scatter(b2)

                        start_gather(j + 1, b2)

                    wait_gather(k)
                    start_scatter(k)
            return carry

        lax.fori_loop(0, (n_j + NBUF - 1) // NBUF, pair, 0)
        for b in range(NBUF):
            @pl.when(n_j >= b + 1)
            def _(b=b):
                wait_scatter(b)

        plsc.subcore_barrier()

        pltpu.sync_copy(acc.at[pl.ds(zbase, ZROWS)], out_hbm.at[pl.ds(zbase, ZROWS)])

        @pl.when(s == NS - 1)
        def _():
            tb = c * SPLIT + NS * ZROWS
            pltpu.sync_copy(acc.at[pl.ds(tb, ZTAIL)], out_hbm.at[pl.ds(tb, ZTAIL)])

    return seg_sum_kernel(data, ids)


@jax.jit
def kernel(data, segment_ids):
    return _sc_segment_sum(data, segment_ids)


# speedup vs baseline: 5.8387x; 1.0211x over previous
import functools

import jax
import jax.numpy as jnp
from jax import lax
from jax.experimental import pallas as pl
from jax.experimental.pallas import tpu as pltpu
from jax.experimental.pallas import tpu_sc as plsc

N_ROWS = 320000
D = 128
N_SEG = 10000
NC = 2
NS = 16
PIECE = 128
N_PIECES = N_ROWS // PIECE
NBUF = 4
HALF = N_SEG // 2
TRASH = HALF
ACC_ROWS = HALF + 8
BS_ITERS = 12
ZROWS = 312
ZTAIL = HALF - ZROWS * NS


def _sc_segment_sum(data, ids):
    mesh = plsc.VectorSubcoreMesh(
        core_axis_name="c", subcore_axis_name="s", num_cores=NC, num_subcores=NS
    )

    @functools.partial(
        pl.kernel,
        out_type=jax.ShapeDtypeStruct((N_SEG, D), jnp.float32),
        mesh=mesh,
        scratch_types=[
            pltpu.VMEM((NBUF, PIECE, D), jnp.float32),
            pltpu.VMEM((NBUF, PIECE), jnp.int32),
            pltpu.VMEM((16,), jnp.int32),
            pltpu.VMEM_SHARED((ACC_ROWS, D), jnp.float32),
        ]
        + [pltpu.SemaphoreType.DMA] * (3 * NBUF + 1),
    )
    def seg_sum_kernel(data_hbm, ids_hbm, out_hbm, db, ib, sb, acc, *sems):
        sem_g = sems[:NBUF]
        sem_i = sems[NBUF : 2 * NBUF]
        sem_s = sems[2 * NBUF : 3 * NBUF]
        sem_z = sems[3 * NBUF]
        c = lax.axis_index("c")
        s = lax.axis_index("s")

        def row0_of(j):
            p = jnp.where(c == 0, s + NS * j, (N_PIECES - 1) - s - NS * j)
            return pl.multiple_of(p * PIECE, PIECE)

        def start_gather(j, b):
            r0 = row0_of(j)
            pltpu.async_copy(data_hbm.at[pl.ds(r0, PIECE)], db.at[b], sem_g[b])
            pltpu.async_copy(ids_hbm.at[pl.ds(r0, PIECE)], ib.at[b], sem_i[b])

        def wait_gather(b):
            pltpu.make_async_copy(
                data_hbm.at[pl.ds(0, PIECE)], db.at[b], sem_g[b]
            ).wait()
            pltpu.make_async_copy(
                ids_hbm.at[pl.ds(0, PIECE)], ib.at[b], sem_i[b]
            ).wait()

        def start_scatter(b):
            pltpu.async_copy(db.at[b], acc.at[ib.at[b]], sem_s[b], add=True)

        def wait_scatter(b):
            pltpu.make_async_copy(db.at[b], acc.at[ib.at[0]], sem_s[b]).wait()

        for j in range(NBUF - 1):
            start_gather(j, j)

        zeros16 = jnp.zeros((16,), jnp.float32)
        zb = NBUF - 1

        def zrow(r, carry):
            def zlane(l, cc):
                db[zb, r, pl.ds(l * 16, 16)] = zeros16
                return cc
            return lax.fori_loop(0, D // 16, zlane, carry)

        lax.fori_loop(0, PIECE, zrow, 0)
        zbase = s * ZROWS
        zcopies = [(zbase, PIECE), (zbase + PIECE, PIECE), (zbase + 256, ZROWS - 256)]
        for off, n in zcopies:
            pltpu.async_copy(db.at[zb, pl.ds(0, n)], acc.at[pl.ds(off, n)], sem_z)

        @pl.when(s == NS - 1)
        def _():
            pltpu.async_copy(
                db.at[zb, pl.ds(0, ZTAIL)],
                acc.at[pl.ds(NS * ZROWS, ZTAIL)],
                sem_z,
            )

        def probe(q):
            pltpu.sync_copy(
                ids_hbm.at[pl.ds(pl.multiple_of(q * PIECE, PIECE), 16)], sb
            )
            v = sb[...]
            return v[0] >= HALF

        def bs_step(_, lohi):
            lo, hi = lohi
            mid = (lo + hi) // 2
            pred = probe(mid)
            return jnp.where(pred, lo, mid), jnp.where(pred, mid, hi)

        pred0 = probe(0)
        _, qb = lax.fori_loop(0, BS_ITERS, bs_step, (0, N_PIECES))
        qb = jnp.where(pred0, 0, qb)
        sp = jnp.maximum(qb - 1, 0)

        for off, n in zcopies:
            pltpu.make_async_copy(
                db.at[zb, pl.ds(0, n)], acc.at[pl.ds(off, n)], sem_z
            ).wait()

        @pl.when(s == NS - 1)
        def _():
            pltpu.make_async_copy(
                db.at[zb, pl.ds(0, ZTAIL)], acc.at[pl.ds(NS * ZROWS, ZTAIL)], sem_z
            ).wait()

        plsc.subcore_barrier()

        n_sc = jnp.where(c == 0, sp + 1, N_PIECES - sp)
        n_j = (n_sc - s + NS - 1) // NS

        def remap(b):
            half = jnp.full((16,), HALF, jnp.int32)
            trash = jnp.full((16,), TRASH, jnp.int32)
            for g in range(PIECE // 16):
                t = ib[b, pl.ds(g * 16, 16)] - c * HALF
                t = jnp.where((t < 0) | (t >= half), trash, t)
                ib[b, pl.ds(g * 16, 16)] = t

        def quad(jj, carry):
            for k in range(NBUF):
                j = jj * NBUF + k
                b3 = (k + NBUF - 1) % NBUF

                @pl.when(j < n_j)
                def _(j=j, k=k, b3=b3):
                    @pl.when(j + (NBUF - 1) < n_j)
                    def _():
                        @pl.when(j >= 1)
                        def _():
                            wait_scatter(b3)

                        start_gather(j + (NBUF - 1), b3)

                    wait_gather(k)
                    remap(k)
                    start_scatter(k)
            return carry

        lax.fori_loop(0, (n_j + NBUF - 1) // NBUF, quad, 0)

        for j in range(NBUF - 1):
            @pl.when(n_j <= j)
            def _(j=j):
                wait_gather(j)

        for b in range(NBUF):
            @pl.when(n_j >= b + 1)
            def _(b=b):
                wait_scatter(b)

        plsc.subcore_barrier()

        pltpu.sync_copy(
            acc.at[pl.ds(zbase, ZROWS)],
            out_hbm.at[pl.ds(c * HALF + zbase, ZROWS)],
        )

        @pl.when(s == NS - 1)
        def _():
            pltpu.sync_copy(
                acc.at[pl.ds(NS * ZROWS, ZTAIL)],
                out_hbm.at[pl.ds(c * HALF + NS * ZROWS, ZTAIL)],
            )

    return seg_sum_kernel(data, ids)


@jax.jit
def kernel(data, segment_ids):
    return _sc_segment_sum(data, segment_ids)


# speedup vs baseline: 5.8694x; 1.0053x over previous
import functools

import jax
import jax.numpy as jnp
from jax import lax
from jax.experimental import pallas as pl
from jax.experimental.pallas import tpu as pltpu
from jax.experimental.pallas import tpu_sc as plsc

N_ROWS = 320000
D = 128
N_SEG = 10000
NC = 2
NS = 16
PIECE = 128
N_PIECES = N_ROWS // PIECE
NBUF = 4
HALF = N_SEG // 2
TRASH = HALF
ACC_ROWS = HALF + 8
BS_ITERS = 12
ZROWS = 312
ZTAIL = HALF - ZROWS * NS
NSS = 4
SUB = PIECE // NSS


def _sc_segment_sum(data, ids):
    mesh = plsc.VectorSubcoreMesh(
        core_axis_name="c", subcore_axis_name="s", num_cores=NC, num_subcores=NS
    )

    @functools.partial(
        pl.kernel,
        out_type=jax.ShapeDtypeStruct((N_SEG, D), jnp.float32),
        mesh=mesh,
        scratch_types=[
            pltpu.VMEM((NBUF, PIECE, D), jnp.float32),
            pltpu.VMEM((NBUF, NSS, PIECE // NSS), jnp.int32),
            pltpu.VMEM((16,), jnp.int32),
            pltpu.VMEM_SHARED((ACC_ROWS, D), jnp.float32),
        ]
        + [pltpu.SemaphoreType.DMA] * (3 * NBUF + 1),
    )
    def seg_sum_kernel(data_hbm, ids_hbm, out_hbm, db, ib, sb, acc, *sems):
        sem_g = sems[:NBUF]
        sem_i = sems[NBUF : 2 * NBUF]
        sem_s = sems[2 * NBUF : 3 * NBUF]
        sem_z = sems[3 * NBUF]
        c = lax.axis_index("c")
        s = lax.axis_index("s")

        def row0_of(j):
            p = jnp.where(c == 0, s + NS * j, (N_PIECES - 1) - s - NS * j)
            return pl.multiple_of(p * PIECE, PIECE)

        def start_gather(j, b):
            r0 = row0_of(j)
            pltpu.async_copy(data_hbm.at[pl.ds(r0, PIECE)], db.at[b], sem_g[b])
            for m in range(NSS):
                pltpu.async_copy(
                    ids_hbm.at[pl.ds(r0 + m * SUB, SUB)], ib.at[b, m], sem_i[b]
                )

        def wait_gather(b):
            pltpu.make_async_copy(
                data_hbm.at[pl.ds(0, PIECE)], db.at[b], sem_g[b]
            ).wait()
            for m in range(NSS):
                pltpu.make_async_copy(
                    ids_hbm.at[pl.ds(0, SUB)], ib.at[b, m], sem_i[b]
                ).wait()

        def start_scatter(b):
            for m in range(NSS):
                pltpu.async_copy(
                    db.at[b, pl.ds(m * SUB, SUB)],
                    acc.at[ib.at[b, m]],
                    sem_s[b],
                    add=True,
                )

        def wait_scatter(b):
            for m in range(NSS):
                pltpu.make_async_copy(
                    db.at[0, pl.ds(0, SUB)], acc.at[ib.at[0, 0]], sem_s[b]
                ).wait()

        for j in range(NBUF - 1):
            start_gather(j, j)

        zeros16 = jnp.zeros((16,), jnp.float32)
        zb = NBUF - 1

        def zrow(r, carry):
            def zlane(l, cc):
                db[zb, r, pl.ds(l * 16, 16)] = zeros16
                return cc
            return lax.fori_loop(0, D // 16, zlane, carry)

        lax.fori_loop(0, PIECE, zrow, 0)
        zbase = s * ZROWS
        zcopies = [(zbase, PIECE), (zbase + PIECE, PIECE), (zbase + 256, ZROWS - 256)]
        for off, n in zcopies:
            pltpu.async_copy(db.at[zb, pl.ds(0, n)], acc.at[pl.ds(off, n)], sem_z)

        @pl.when(s == NS - 1)
        def _():
            pltpu.async_copy(
                db.at[zb, pl.ds(0, ZTAIL)],
                acc.at[pl.ds(NS * ZROWS, ZTAIL)],
                sem_z,
            )

        def probe(q):
            pltpu.sync_copy(
                ids_hbm.at[pl.ds(pl.multiple_of(q * PIECE, PIECE), 16)], sb
            )
            v = sb[...]
            return v[0] >= HALF

        def bs_step(_, lohi):
            lo, hi = lohi
            mid = (lo + hi) // 2
            pred = probe(mid)
            return jnp.where(pred, lo, mid), jnp.where(pred, mid, hi)

        pred0 = probe(0)
        _, qb = lax.fori_loop(0, BS_ITERS, bs_step, (0, N_PIECES))
        qb = jnp.where(pred0, 0, qb)
        sp = jnp.maximum(qb - 1, 0)

        for off, n in zcopies:
            pltpu.make_async_copy(
                db.at[zb, pl.ds(0, n)], acc.at[pl.ds(off, n)], sem_z
            ).wait()

        @pl.when(s == NS - 1)
        def _():
            pltpu.make_async_copy(
                db.at[zb, pl.ds(0, ZTAIL)], acc.at[pl.ds(NS * ZROWS, ZTAIL)], sem_z
            ).wait()

        plsc.subcore_barrier()

        n_sc = jnp.where(c == 0, sp + 1, N_PIECES - sp)
        n_j = (n_sc - s + NS - 1) // NS

        def remap(b):
            half = jnp.full((16,), HALF, jnp.int32)
            trash = jnp.full((16,), TRASH, jnp.int32)
            for m in range(NSS):
                for g in range(SUB // 16):
                    t = ib[b, m, pl.ds(g * 16, 16)] - c * HALF
                    t = jnp.where((t < 0) | (t >= half), trash, t)
                    ib[b, m, pl.ds(g * 16, 16)] = t

        def quad(jj, carry):
            for k in range(NBUF):
                j = jj * NBUF + k
                b3 = (k + NBUF - 1) % NBUF

                @pl.when(j < n_j)
                def _(j=j, k=k, b3=b3):
                    @pl.when(j + (NBUF - 1) < n_j)
                    def _():
                        @pl.when(j >= 1)
                        def _():
                            wait_scatter(b3)

                        start_gather(j + (NBUF - 1), b3)

                    wait_gather(k)
                    remap(k)
                    start_scatter(k)
            return carry

        lax.fori_loop(0, (n_j + NBUF - 1) // NBUF, quad, 0)

        for j in range(NBUF - 1):
            @pl.when(n_j <= j)
            def _(j=j):
                wait_gather(j)

        for b in range(NBUF):
            @pl.when(n_j >= b + 1)
            def _(b=b):
                wait_scatter(b)

        plsc.subcore_barrier()

        pltpu.sync_copy(
            acc.at[pl.ds(zbase, ZROWS)],
            out_hbm.at[pl.ds(c * HALF + zbase, ZROWS)],
        )

        @pl.when(s == NS - 1)
        def _():
            pltpu.sync_copy(
                acc.at[pl.ds(NS * ZROWS, ZTAIL)],
                out_hbm.at[pl.ds(c * HALF + NS * ZROWS, ZTAIL)],
            )

    return seg_sum_kernel(data, ids)


@jax.jit
def kernel(data, segment_ids):
    return _sc_segment_sum(data, segment_ids)


# speedup vs baseline: 5.8989x; 1.0050x over previous
import functools

import jax
import jax.numpy as jnp
from jax import lax
from jax.experimental import pallas as pl
from jax.experimental.pallas import tpu as pltpu
from jax.experimental.pallas import tpu_sc as plsc

N_ROWS = 320000
D = 128
N_SEG = 10000
NC = 2
NS = 16
PIECE = 128
N_PIECES = N_ROWS // PIECE
NBUF = 4
HALF = N_SEG // 2
TRASH = HALF
ACC_ROWS = HALF + 8
BS_ITERS = 12
ZROWS = 312
ZTAIL = HALF - ZROWS * NS
NSS = 4
SUB = PIECE // NSS


def _sc_segment_sum(data, ids):
    mesh = plsc.VectorSubcoreMesh(
        core_axis_name="c", subcore_axis_name="s", num_cores=NC, num_subcores=NS
    )

    @functools.partial(
        pl.kernel,
        out_type=jax.ShapeDtypeStruct((N_SEG, D), jnp.float32),
        mesh=mesh,
        scratch_types=[
            pltpu.VMEM((NBUF, PIECE, D), jnp.float32),
            pltpu.VMEM((NBUF, NSS, PIECE // NSS), jnp.int32),
            pltpu.VMEM((16,), jnp.int32),
            pltpu.VMEM_SHARED((ACC_ROWS, D), jnp.float32),
        ]
        + [pltpu.SemaphoreType.DMA] * (3 * NBUF + 1),
    )
    def seg_sum_kernel(data_hbm, ids_hbm, out_hbm, db, ib, sb, acc, *sems):
        sem_g = sems[:NBUF]
        sem_i = sems[NBUF : 2 * NBUF]
        sem_s = sems[2 * NBUF : 3 * NBUF]
        sem_z = sems[3 * NBUF]
        c = lax.axis_index("c")
        s = lax.axis_index("s")

        def row0_of(j):
            p = jnp.where(c == 0, s + NS * j, (N_PIECES - 1) - s - NS * j)
            return pl.multiple_of(p * PIECE, PIECE)

        def start_gather(j, b):
            r0 = row0_of(j)
            half_p = PIECE // 2
            pltpu.async_copy(
                data_hbm.at[pl.ds(r0, half_p)], db.at[b, pl.ds(0, half_p)], sem_g[b]
            )
            pltpu.async_copy(
                data_hbm.at[pl.ds(r0 + half_p, half_p)],
                db.at[b, pl.ds(half_p, half_p)],
                sem_g[b],
            )
            for m in range(NSS):
                pltpu.async_copy(
                    ids_hbm.at[pl.ds(r0 + m * SUB, SUB)], ib.at[b, m], sem_i[b]
                )

        def wait_gather(b):
            for _h in range(2):
                pltpu.make_async_copy(
                    data_hbm.at[pl.ds(0, PIECE // 2)],
                    db.at[b, pl.ds(0, PIECE // 2)],
                    sem_g[b],
                ).wait()
            for m in range(NSS):
                pltpu.make_async_copy(
                    ids_hbm.at[pl.ds(0, SUB)], ib.at[b, m], sem_i[b]
                ).wait()

        def start_scatter(b):
            for m in range(NSS):
                pltpu.async_copy(
                    db.at[b, pl.ds(m * SUB, SUB)],
                    acc.at[ib.at[b, m]],
                    sem_s[b],
                    add=True,
                )

        def wait_scatter(b):
            for m in range(NSS):
                pltpu.make_async_copy(
                    db.at[0, pl.ds(0, SUB)], acc.at[ib.at[0, 0]], sem_s[b]
                ).wait()

        for j in range(NBUF - 1):
            start_gather(j, j)

        zeros16 = jnp.zeros((16,), jnp.float32)
        zb = NBUF - 1

        def zrow(r, carry):
            def zlane(l, cc):
                db[zb, r, pl.ds(l * 16, 16)] = zeros16
                return cc
            return lax.fori_loop(0, D // 16, zlane, carry)

        lax.fori_loop(0, PIECE, zrow, 0)
        zbase = s * ZROWS
        zcopies = [(zbase, PIECE), (zbase + PIECE, PIECE), (zbase + 256, ZROWS - 256)]
        for off, n in zcopies:
            pltpu.async_copy(db.at[zb, pl.ds(0, n)], acc.at[pl.ds(off, n)], sem_z)

        @pl.when(s == NS - 1)
        def _():
            pltpu.async_copy(
                db.at[zb, pl.ds(0, ZTAIL)],
                acc.at[pl.ds(NS * ZROWS, ZTAIL)],
                sem_z,
            )

        def probe(q):
            pltpu.sync_copy(
                ids_hbm.at[pl.ds(pl.multiple_of(q * PIECE, PIECE), 16)], sb
            )
            v = sb[...]
            return v[0] >= HALF

        def bs_step(_, lohi):
            lo, hi = lohi
            mid = (lo + hi) // 2
            pred = probe(mid)
            return jnp.where(pred, lo, mid), jnp.where(pred, mid, hi)

        pred0 = probe(0)
        _, qb = lax.fori_loop(0, BS_ITERS, bs_step, (0, N_PIECES))
        qb = jnp.where(pred0, 0, qb)
        sp = jnp.maximum(qb - 1, 0)

        for off, n in zcopies:
            pltpu.make_async_copy(
                db.at[zb, pl.ds(0, n)], acc.at[pl.ds(off, n)], sem_z
            ).wait()

        @pl.when(s == NS - 1)
        def _():
            pltpu.make_async_copy(
                db.at[zb, pl.ds(0, ZTAIL)], acc.at[pl.ds(NS * ZROWS, ZTAIL)], sem_z
            ).wait()

        plsc.subcore_barrier()

        n_sc = jnp.where(c == 0, sp + 1, N_PIECES - sp)
        n_j = (n_sc - s + NS - 1) // NS

        def remap(b):
            half = jnp.full((16,), HALF, jnp.int32)
            trash = jnp.full((16,), TRASH, jnp.int32)
            for m in range(NSS):
                for g in range(SUB // 16):
                    t = ib[b, m, pl.ds(g * 16, 16)] - c * HALF
                    t = jnp.where((t < 0) | (t >= half), trash, t)
                    ib[b, m, pl.ds(g * 16, 16)] = t

        def quad(jj, carry):
            for k in range(NBUF):
                j = jj * NBUF + k
                b3 = (k + NBUF - 1) % NBUF

                @pl.when(j < n_j)
                def _(j=j, k=k, b3=b3):
                    @pl.when(j + (NBUF - 1) < n_j)
                    def _():
                        @pl.when(j >= 1)
                        def _():
                            wait_scatter(b3)

                        start_gather(j + (NBUF - 1), b3)

                    wait_gather(k)
                    remap(k)
                    start_scatter(k)
            return carry

        lax.fori_loop(0, (n_j + NBUF - 1) // NBUF, quad, 0)

        for j in range(NBUF - 1):
            @pl.when(n_j <= j)
            def _(j=j):
                wait_gather(j)

        for b in range(NBUF):
            @pl.when(n_j >= b + 1)
            def _(b=b):
                wait_scatter(b)

        plsc.subcore_barrier()

        pltpu.sync_copy(
            acc.at[pl.ds(zbase, ZROWS)],
            out_hbm.at[pl.ds(c * HALF + zbase, ZROWS)],
        )

        @pl.when(s == NS - 1)
        def _():
            pltpu.sync_copy(
                acc.at[pl.ds(NS * ZROWS, ZTAIL)],
                out_hbm.at[pl.ds(c * HALF + NS * ZROWS, ZTAIL)],
            )

    return seg_sum_kernel(data, ids)


@jax.jit
def kernel(data, segment_ids):
    return _sc_segment_sum(data, segment_ids)


# speedup vs baseline: 6.0325x; 1.0226x over previous
import functools

import jax
import jax.numpy as jnp
from jax import lax
from jax.experimental import pallas as pl
from jax.experimental.pallas import tpu as pltpu
from jax.experimental.pallas import tpu_sc as plsc

N_ROWS = 320000
D = 128
N_SEG = 10000
NC = 2
NS = 16
PIECE = 128
N_PIECES = N_ROWS // PIECE
NBUF = 4
HALF = N_SEG // 2
TRASH = HALF
ACC_ROWS = HALF + 8
ZROWS = 312
ZTAIL = HALF - ZROWS * NS
NSS = 4
SUB = PIECE // NSS


def _sc_segment_sum(data, ids):
    mesh = plsc.VectorSubcoreMesh(
        core_axis_name="c", subcore_axis_name="s", num_cores=NC, num_subcores=NS
    )

    @functools.partial(
        pl.kernel,
        out_type=jax.ShapeDtypeStruct((N_SEG, D), jnp.float32),
        mesh=mesh,
        scratch_types=[
            pltpu.VMEM((NBUF, PIECE, D), jnp.float32),
            pltpu.VMEM((NBUF, NSS, PIECE // NSS), jnp.int32),
            pltpu.VMEM((16, 16), jnp.int32),
            pltpu.VMEM_SHARED((ACC_ROWS, D), jnp.float32),
        ]
        + [pltpu.SemaphoreType.DMA] * (3 * NBUF + 2),
    )
    def seg_sum_kernel(data_hbm, ids_hbm, out_hbm, db, ib, sb, acc, *sems):
        sem_g = sems[:NBUF]
        sem_i = sems[NBUF : 2 * NBUF]
        sem_s = sems[2 * NBUF : 3 * NBUF]
        sem_z = sems[3 * NBUF]
        sem_p = sems[3 * NBUF + 1]
        c = lax.axis_index("c")
        s = lax.axis_index("s")

        def row0_of(j):
            p = jnp.where(c == 0, s + NS * j, (N_PIECES - 1) - s - NS * j)
            return pl.multiple_of(p * PIECE, PIECE)

        def start_gather(j, b):
            r0 = row0_of(j)
            half_p = PIECE // 2
            pltpu.async_copy(
                data_hbm.at[pl.ds(r0, half_p)], db.at[b, pl.ds(0, half_p)], sem_g[b]
            )
            pltpu.async_copy(
                data_hbm.at[pl.ds(r0 + half_p, half_p)],
                db.at[b, pl.ds(half_p, half_p)],
                sem_g[b],
            )
            for m in range(NSS):
                pltpu.async_copy(
                    ids_hbm.at[pl.ds(r0 + m * SUB, SUB)], ib.at[b, m], sem_i[b]
                )

        def wait_gather(b):
            for _h in range(2):
                pltpu.make_async_copy(
                    data_hbm.at[pl.ds(0, PIECE // 2)],
                    db.at[b, pl.ds(0, PIECE // 2)],
                    sem_g[b],
                ).wait()
            for m in range(NSS):
                pltpu.make_async_copy(
                    ids_hbm.at[pl.ds(0, SUB)], ib.at[b, m], sem_i[b]
                ).wait()

        def start_scatter(b):
            for m in range(NSS):
                pltpu.async_copy(
                    db.at[b, pl.ds(m * SUB, SUB)],
                    acc.at[ib.at[b, m]],
                    sem_s[b],
                    add=True,
                )

        def wait_scatter(b):
            for m in range(NSS):
                pltpu.make_async_copy(
                    db.at[0, pl.ds(0, SUB)], acc.at[ib.at[0, 0]], sem_s[b]
                ).wait()

        for j in range(NBUF - 1):
            start_gather(j, j)

        zeros16 = jnp.zeros((16,), jnp.float32)
        zb = NBUF - 1

        def zrow(r, carry):
            def zlane(l, cc):
                db[zb, r, pl.ds(l * 16, 16)] = zeros16
                return cc
            return lax.fori_loop(0, D // 16, zlane, carry)

        lax.fori_loop(0, PIECE, zrow, 0)
        zbase = s * ZROWS
        zcopies = [(zbase, PIECE), (zbase + PIECE, PIECE), (zbase + 256, ZROWS - 256)]
        for off, n in zcopies:
            pltpu.async_copy(db.at[zb, pl.ds(0, n)], acc.at[pl.ds(off, n)], sem_z)

        @pl.when(s == NS - 1)
        def _():
            pltpu.async_copy(
                db.at[zb, pl.ds(0, ZTAIL)],
                acc.at[pl.ds(NS * ZROWS, ZTAIL)],
                sem_z,
            )

        def probe_round(lo, step):
            for i in range(16):
                q = jnp.minimum(lo + i * step, N_PIECES - 1)
                pltpu.async_copy(
                    ids_hbm.at[pl.ds(pl.multiple_of(q * PIECE, PIECE), 16)],
                    sb.at[i],
                    sem_p,
                )
            for i in range(16):
                pltpu.make_async_copy(
                    ids_hbm.at[pl.ds(0, 16)], sb.at[i], sem_p
                ).wait()
            nf = jnp.int32(0)
            for i in range(16):
                v = sb[i, pl.ds(0, 16)]
                nf = nf + jnp.where(v[0] < HALF, 1, 0).astype(jnp.int32)
            return nf

        lo = jnp.int32(0)
        nf1 = probe_round(lo, 157)
        pred0 = nf1 == 0
        lo = lo + (jnp.maximum(nf1, 1) - 1) * 157
        nf2 = probe_round(lo, 10)
        lo = lo + (jnp.maximum(nf2, 1) - 1) * 10
        nf3 = probe_round(lo, 1)
        qb = jnp.minimum(lo + nf3, N_PIECES)
        qb = jnp.where(pred0, 0, qb)
        sp = jnp.maximum(qb - 1, 0)

        for off, n in zcopies:
            pltpu.make_async_copy(
                db.at[zb, pl.ds(0, n)], acc.at[pl.ds(off, n)], sem_z
            ).wait()

        @pl.when(s == NS - 1)
        def _():
            pltpu.make_async_copy(
                db.at[zb, pl.ds(0, ZTAIL)], acc.at[pl.ds(NS * ZROWS, ZTAIL)], sem_z
            ).wait()

        plsc.subcore_barrier()

        n_sc = jnp.where(c == 0, sp + 1, N_PIECES - sp)
        n_j = (n_sc - s + NS - 1) // NS

        def remap(b):
            half = jnp.full((16,), HALF, jnp.int32)
            trash = jnp.full((16,), TRASH, jnp.int32)
            for m in range(NSS):
                for g in range(SUB // 16):
                    t = ib[b, m, pl.ds(g * 16, 16)] - c * HALF
                    t = jnp.where((t < 0) | (t >= half), trash, t)
                    ib[b, m, pl.ds(g * 16, 16)] = t

        def quad(jj, carry):
            for k in range(NBUF):
                j = jj * NBUF + k
                b3 = (k + NBUF - 1) % NBUF

                @pl.when(j < n_j)
                def _(j=j, k=k, b3=b3):
                    @pl.when(j + (NBUF - 1) < n_j)
                    def _():
                        @pl.when(j >= 1)
                        def _():
                            wait_scatter(b3)

                        start_gather(j + (NBUF - 1), b3)

                    wait_gather(k)
                    remap(k)
                    start_scatter(k)
            return carry

        lax.fori_loop(0, (n_j + NBUF - 1) // NBUF, quad, 0)

        for j in range(NBUF - 1):
            @pl.when(n_j <= j)
            def _(j=j):
                wait_gather(j)

        for b in range(NBUF):
            @pl.when(n_j >= b + 1)
            def _(b=b):
                wait_scatter(b)

        plsc.subcore_barrier()

        pltpu.sync_copy(
            acc.at[pl.ds(zbase, ZROWS)],
            out_hbm.at[pl.ds(c * HALF + zbase, ZROWS)],
        )

        @pl.when(s == NS - 1)
        def _():
            pltpu.sync_copy(
                acc.at[pl.ds(NS * ZROWS, ZTAIL)],
                out_hbm.at[pl.ds(c * HALF + NS * ZROWS, ZTAIL)],
            )

    return seg_sum_kernel(data, ids)


@jax.jit
def kernel(data, segment_ids):
    return _sc_segment_sum(data, segment_ids)
